# Initial kernel scaffold; baseline (speedup 1.0000x reference)
import jax
import jax.numpy as jnp
from jax.experimental import pallas as pl


def kernel(edge_index, input_time, input_events1, event_annotation, pos_vec, emb_table, W_in, b_in, attn_W1, attn_b1, attn_W2, attn_b2, node_W1, node_b1, node_W2, node_b2, lstm_Wih, lstm_bih, lstm_Whh, lstm_bhh, model_W, model_b, time_W, time_b, event_W, event_b):
    raise NotImplementedError("write your pallas kernel here")


# speedup vs baseline: 9.8814x; 9.8814x over previous
import jax
import jax.numpy as jnp
from jax import lax
from jax.experimental import pallas as pl
from jax.experimental.pallas import tpu as pltpu
from jax.experimental.pallas import tpu_sc as plsc

H = 128
NP = 10240
EP = 163840
F32 = jnp.float32
I32 = jnp.int32


def _elu(x):
    return jnp.where(x > 0, x, jnp.exp(jnp.minimum(x, 0.0)) - 1.0)


def _sigmoid(x):
    return jax.nn.sigmoid(x)


def _mesh():
    return plsc.VectorSubcoreMesh(core_axis_name="c", subcore_axis_name="s")


def _prologue_body(time_ref, ev_ref, pos_ref, emb_ref, WembT_ref, wev_ref,
                   WteT_ref, bin_ref, projT_ref, base_ref):
    projT_ref[...] = jnp.dot(emb_ref[...], WembT_ref[...],
                             preferred_element_type=F32)
    pos = pos_ref[...]
    col = lax.broadcasted_iota(I32, (NP, H), 1)
    is_even = (col % 2) == 0
    for t in range(2):
        tt = time_ref[:, t:t + 1]
        r = tt / pos
        te = jnp.where(is_even, jnp.sin(r), jnp.cos(r))
        m = (ev_ref[:, t:t + 1] != 0).astype(F32)
        te = te * m
        base = (jnp.dot(te, WteT_ref[...], preferred_element_type=F32)
                + ev_ref[:, t:t + 1] * wev_ref[...] + bin_ref[...])
        base_ref[:, t, :] = base


def _tc_prologue(time_p, ev_p, pos2, emb_pad, WembT, wev2, WteT, bin2):
    return pl.pallas_call(
        _prologue_body,
        out_shape=(jax.ShapeDtypeStruct((1024, H), F32),
                   jax.ShapeDtypeStruct((NP, 2, H), F32)),
    )(time_p, ev_p, pos2, emb_pad, WembT, wev2, WteT, bin2)


def _sc_embed_body(ann_hbm, base_hbm, projT_hbm, out_hbm, idx_v, rows_v,
                   acc_v, sem):
    c = lax.axis_index("c")
    s = lax.axis_index("s")
    w = c * 16 + s
    row0 = w * 640
    pltpu.sync_copy(ann_hbm.at[w], idx_v)

    def chunk(j, carry):
        base = row0 + j * 128
        pltpu.async_copy(projT_hbm.at[idx_v.at[j]], rows_v, sem).wait()
        pltpu.sync_copy(base_hbm.at[pl.ds(base, 128)], acc_v)

        def add_row(r, carry2):
            for kk in range(8):
                sl = pl.ds(kk * 16, 16)
                acc_v[r, sl] = acc_v[r, sl] + rows_v[r, sl]
            return carry2
        lax.fori_loop(0, 128, add_row, 0)
        pltpu.sync_copy(acc_v, out_hbm.at[pl.ds(base, 128)])
        return carry
    lax.fori_loop(0, 5, chunk, 0)


def _sc_embed(ann3, base_flat, projT):
    kern = pl.kernel(
        _sc_embed_body,
        out_type=jax.ShapeDtypeStruct((NP * 2, H), F32),
        mesh=_mesh(),
        scratch_types=[
            pltpu.VMEM((5, 128), I32),
            pltpu.VMEM((128, H), F32),
            pltpu.VMEM((128, H), F32),
            pltpu.SemaphoreType.DMA,
        ],
    )
    return kern(ann3, base_flat, projT)


def _sc_edge_gather_body(nf_hbm, s0_hbm, s1_hbm, d0_hbm, d1_hbm, dc_hbm,
                         ones_hbm, zc_hbm,
                         gs0_hbm, gs1_hbm, gd0_hbm, gd1_hbm, cnt_hbm,
                         idx_v, buf, ones_v, acc_sh, sem):
    c = lax.axis_index("c")
    s = lax.axis_index("s")
    w = c * 16 + s
    pltpu.sync_copy(zc_hbm.at[pl.ds(s * 640, 640)],
                    acc_sh.at[pl.ds(s * 640, 640)])
    pltpu.sync_copy(ones_hbm, ones_v)
    plsc.subcore_barrier()
    e0 = w * 5120

    for (src_idx_hbm, out_hbm) in ((s0_hbm, gs0_hbm), (s1_hbm, gs1_hbm),
                                   (d0_hbm, gd0_hbm), (d1_hbm, gd1_hbm)):
        pltpu.sync_copy(src_idx_hbm.at[w], idx_v)

        def body(j, carry):
            row = e0 + j * 128
            pltpu.async_copy(nf_hbm.at[idx_v.at[j]], buf, sem).wait()
            pltpu.sync_copy(buf, out_hbm.at[pl.ds(row, 128)])
            return carry
        lax.fori_loop(0, 40, body, 0)

    pltpu.sync_copy(dc_hbm.at[w], idx_v)

    def cbody(j, carry):
        pltpu.sync_copy(ones_v, acc_sh.at[idx_v.at[j]], add=True)
        return carry
    lax.fori_loop(0, 40, cbody, 0)
    plsc.subcore_barrier()
    pltpu.sync_copy(acc_sh.at[pl.ds(s * 640, 640)],
                    cnt_hbm.at[c, pl.ds(s * 640, 640)])


def _sc_edge_gather(nf_flat, s0, s1, d0, d1, dc, ones128, zc):
    kern = pl.kernel(
        _sc_edge_gather_body,
        out_type=(jax.ShapeDtypeStruct((EP, H), F32),
                  jax.ShapeDtypeStruct((EP, H), F32),
                  jax.ShapeDtypeStruct((EP, H), F32),
                  jax.ShapeDtypeStruct((EP, H), F32),
                  jax.ShapeDtypeStruct((2, NP, H), F32)),
        mesh=_mesh(),
        scratch_types=[
            pltpu.VMEM((40, 128), I32),
            pltpu.VMEM((128, H), F32),
            pltpu.VMEM((128, H), F32),
            pltpu.VMEM_SHARED((NP, H), F32),
            pltpu.SemaphoreType.DMA,
        ],
    )
    return kern(nf_flat, s0, s1, d0, d1, dc, ones128, zc)


B_E = 1024


def _edge_mlp_body(gs0_ref, gs1_ref, gd0_ref, gd1_ref, W1LT_ref, W1RT_ref,
                   b1_ref, W2T_ref, b2_ref, Wn1T_ref, b_ref):
    for t, (gs_ref, gd_ref) in enumerate(((gs0_ref, gd0_ref),
                                          (gs1_ref, gd1_ref))):
        h = (jnp.dot(gs_ref[...], W1LT_ref[...], preferred_element_type=F32)
             + jnp.dot(gd_ref[...], W1RT_ref[...], preferred_element_type=F32)
             + b1_ref[...])
        h = _elu(h)
        a = _elu(jnp.dot(h, W2T_ref[...], preferred_element_type=F32)
                 + b2_ref[...])
        b_ref[t, :, :] = jnp.dot(a, Wn1T_ref[...], preferred_element_type=F32)


def _tc_edge_mlp(gs0, gs1, gd0, gd1, W1LT, W1RT, b1, W2T, b2, Wn1T):
    nb = EP // B_E
    edge_spec = pl.BlockSpec((B_E, H), lambda i: (i, 0))
    return pl.pallas_call(
        _edge_mlp_body,
        grid=(nb,),
        in_specs=[
            edge_spec, edge_spec, edge_spec, edge_spec,
            pl.BlockSpec((H, 2 * H), lambda i: (0, 0)),
            pl.BlockSpec((H, 2 * H), lambda i: (0, 0)),
            pl.BlockSpec((1, 2 * H), lambda i: (0, 0)),
            pl.BlockSpec((2 * H, 2 * H), lambda i: (0, 0)),
            pl.BlockSpec((1, 2 * H), lambda i: (0, 0)),
            pl.BlockSpec((2 * H, H), lambda i: (0, 0)),
        ],
        out_specs=pl.BlockSpec((2, B_E, H), lambda i: (0, i, 0)),
        out_shape=jax.ShapeDtypeStruct((2, EP, H), F32),
    )(gs0, gs1, gd0, gd1, W1LT, W1RT, b1, W2T, b2, Wn1T)


def _sc_scatter_body(b_hbm, dst_hbm, zs_hbm, sums_hbm, idx, rows, acc_sh, sem):
    c = lax.axis_index("c")
    s = lax.axis_index("s")
    pltpu.sync_copy(zs_hbm.at[pl.ds(s * 640, 640)],
                    acc_sh.at[pl.ds(s * 640, 640)])
    pltpu.sync_copy(dst_hbm.at[s], idx)
    plsc.subcore_barrier()
    e0 = s * 10240

    def body(j, carry):
        pltpu.sync_copy(b_hbm.at[c, pl.ds(e0 + j * 128, 128)], rows)
        pltpu.sync_copy(rows, acc_sh.at[idx.at[j]], add=True)
        return carry
    lax.fori_loop(0, 80, body, 0)
    plsc.subcore_barrier()
    pltpu.sync_copy(acc_sh.at[pl.ds(s * 640, 640)],
                    sums_hbm.at[c, pl.ds(s * 640, 640)])


def _sc_scatter(b2, dst16, zs):
    kern = pl.kernel(
        _sc_scatter_body,
        out_type=jax.ShapeDtypeStruct((2, NP, H), F32),
        mesh=_mesh(),
        scratch_types=[
            pltpu.VMEM((80, 128), I32),
            pltpu.VMEM((128, H), F32),
            pltpu.VMEM_SHARED((NP, H), F32),
            pltpu.SemaphoreType.DMA,
        ],
    )
    return kern(b2, dst16, zs)


B_N = 1024


def _epilogue_body(sums_ref, cnt_ref, nb1_ref, Wn2T_ref, nb2_ref, WihT_ref,
                   WhhT_ref, bihh_ref, hWT_ref, hb_ref, out_ref):
    cvec = cnt_ref[0] + cnt_ref[1]
    den = jnp.maximum(cvec[:, 0:1], 1.0)
    xs = []
    for t in range(2):
        x1 = sums_ref[t] / den + nb1_ref[...]
        nf2 = _elu(jnp.dot(_elu(x1), Wn2T_ref[...],
                           preferred_element_type=F32) + nb2_ref[...])
        xs.append(nf2)
    bihh = bihh_ref[...]
    g1 = jnp.dot(xs[0], WihT_ref[...], preferred_element_type=F32) + bihh
    c1 = _sigmoid(g1[:, 0:H]) * jnp.tanh(g1[:, 2 * H:3 * H])
    h1 = _sigmoid(g1[:, 3 * H:4 * H]) * jnp.tanh(c1)
    g2 = (jnp.dot(xs[1], WihT_ref[...], preferred_element_type=F32)
          + jnp.dot(h1, WhhT_ref[...], preferred_element_type=F32) + bihh)
    c2 = (_sigmoid(g2[:, H:2 * H]) * c1
          + _sigmoid(g2[:, 0:H]) * jnp.tanh(g2[:, 2 * H:3 * H]))
    h2 = _sigmoid(g2[:, 3 * H:4 * H]) * jnp.tanh(c2)
    out_ref[...] = (jnp.dot(h2, hWT_ref[...], preferred_element_type=F32)
                    + hb_ref[...])


def _tc_epilogue(sums, cnt, nb1, Wn2T, nb2, WihT, WhhT, bihh, hWT, hb):
    nb = NP // B_N
    return pl.pallas_call(
        _epilogue_body,
        grid=(nb,),
        in_specs=[
            pl.BlockSpec((2, B_N, H), lambda i: (0, i, 0)),
            pl.BlockSpec((2, B_N, H), lambda i: (0, i, 0)),
            pl.BlockSpec((1, H), lambda i: (0, 0)),
            pl.BlockSpec((H, H), lambda i: (0, 0)),
            pl.BlockSpec((1, H), lambda i: (0, 0)),
            pl.BlockSpec((H, 4 * H), lambda i: (0, 0)),
            pl.BlockSpec((H, 4 * H), lambda i: (0, 0)),
            pl.BlockSpec((1, 4 * H), lambda i: (0, 0)),
            pl.BlockSpec((H, H), lambda i: (0, 0)),
            pl.BlockSpec((1, H), lambda i: (0, 0)),
        ],
        out_specs=pl.BlockSpec((B_N, H), lambda i: (i, 0)),
        out_shape=jax.ShapeDtypeStruct((NP, H), F32),
    )(sums, cnt, nb1, Wn2T, nb2, WihT, WhhT, bihh, hWT, hb)


def kernel(edge_index, input_time, input_events1, event_annotation, pos_vec,
           emb_table, W_in, b_in, attn_W1, attn_b1, attn_W2, attn_b2,
           node_W1, node_b1, node_W2, node_b2, lstm_Wih, lstm_bih, lstm_Whh,
           lstm_bhh, model_W, model_b, time_W, time_b, event_W, event_b):
    N, T = input_time.shape
    E = edge_index.shape[1]

    W_emb = jnp.pad(W_in[:, :H - 1], ((0, 0), (0, 1)))
    wev2 = W_in[:, H - 1].reshape(1, H)
    WteT = W_in[:, H:].T
    W1LT = attn_W1[:, :H].T
    W1RT = attn_W1[:, H:].T
    emb_pad = jnp.pad(emb_table, ((0, 1024 - emb_table.shape[0]), (0, 1)))
    headW = jnp.concatenate(
        [model_W, time_W, event_W, jnp.zeros((H - 3, H), F32)], axis=0)
    headb = jnp.concatenate(
        [model_b, time_b, event_b, jnp.zeros((H - 3,), F32)]).reshape(1, H)

    src = jnp.concatenate(
        [edge_index[0].astype(I32), jnp.zeros((EP - E,), I32)])
    dst = jnp.concatenate(
        [edge_index[1].astype(I32), jnp.full((EP - E,), N, I32)])
    time_p = jnp.pad(input_time, ((0, NP - N), (0, 0)))
    ev_p = jnp.pad(input_events1, ((0, NP - N), (0, 0)))
    ann3 = jnp.pad(event_annotation.astype(I32),
                   ((0, NP - N), (0, 0))).reshape(32, 5, 128)

    projT, base = _tc_prologue(time_p, ev_p, pos_vec.reshape(1, H), emb_pad,
                               W_emb.T, wev2, WteT, b_in.reshape(1, H))
    nf_flat = _sc_embed(ann3, base.reshape(NP * 2, H), projT)

    gs0, gs1, gd0, gd1, cnt = _sc_edge_gather(
        nf_flat,
        (src * 2).reshape(32, 40, 128), (src * 2 + 1).reshape(32, 40, 128),
        (dst * 2).reshape(32, 40, 128), (dst * 2 + 1).reshape(32, 40, 128),
        dst.reshape(32, 40, 128),
        jnp.ones((128, H), F32), jnp.zeros((NP, H), F32))

    b2 = _tc_edge_mlp(gs0, gs1, gd0, gd1, W1LT, W1RT,
                      attn_b1.reshape(1, 2 * H),
                      attn_W2.T, attn_b2.reshape(1, 2 * H), node_W1.T)

    sums = _sc_scatter(b2, dst.reshape(16, 80, 128), jnp.zeros((NP, H), F32))

    heads = _tc_epilogue(sums, cnt, node_b1.reshape(1, H), node_W2.T,
                         node_b2.reshape(1, H), lstm_Wih.T, lstm_Whh.T,
                         (lstm_bih + lstm_bhh).reshape(1, 4 * H),
                         headW.T, headb)
    return heads[:N, 0:1], heads[:N, 1:2], heads[:N, 2:3]


# speedup vs baseline: 10.7953x; 1.0925x over previous
import jax
import jax.numpy as jnp
from jax import lax
from jax.experimental import pallas as pl
from jax.experimental.pallas import tpu as pltpu
from jax.experimental.pallas import tpu_sc as plsc

H = 128
NP = 10240
EP = 163840
F32 = jnp.float32
I32 = jnp.int32


def _elu(x):
    return jnp.where(x > 0, x, jnp.exp(jnp.minimum(x, 0.0)) - 1.0)


def _sigmoid(x):
    return jax.nn.sigmoid(x)


def _mesh():
    return plsc.VectorSubcoreMesh(core_axis_name="c", subcore_axis_name="s")


def _prologue_body(time_ref, ev_ref, pos_ref, emb_ref, WembT_ref, wev_ref,
                   WteT_ref, bin_ref, projT_ref, base_ref):
    projT_ref[...] = jnp.dot(emb_ref[...], WembT_ref[...],
                             preferred_element_type=F32)
    pos = pos_ref[...]
    col = lax.broadcasted_iota(I32, (NP, H), 1)
    is_even = (col % 2) == 0
    for t in range(2):
        tt = time_ref[:, t:t + 1]
        r = tt / pos
        te = jnp.where(is_even, jnp.sin(r), jnp.cos(r))
        m = (ev_ref[:, t:t + 1] != 0).astype(F32)
        te = te * m
        base = (jnp.dot(te, WteT_ref[...], preferred_element_type=F32)
                + ev_ref[:, t:t + 1] * wev_ref[...] + bin_ref[...])
        base_ref[:, t, :] = base


def _tc_prologue(time_p, ev_p, pos2, emb_pad, WembT, wev2, WteT, bin2):
    return pl.pallas_call(
        _prologue_body,
        out_shape=(jax.ShapeDtypeStruct((1024, H), F32),
                   jax.ShapeDtypeStruct((NP, 2, H), F32)),
    )(time_p, ev_p, pos2, emb_pad, WembT, wev2, WteT, bin2)


def _sc_embed_body(ann_hbm, base_hbm, projT_hbm, out_hbm, idx_v, rows_v,
                   acc_v, sem):
    c = lax.axis_index("c")
    s = lax.axis_index("s")
    w = c * 16 + s
    row0 = w * 640
    pltpu.sync_copy(ann_hbm.at[w], idx_v)

    def chunk(j, carry):
        base = row0 + j * 128
        pltpu.async_copy(projT_hbm.at[idx_v.at[j]], rows_v, sem).wait()
        pltpu.sync_copy(base_hbm.at[pl.ds(base, 128)], acc_v)

        def add_row(r, carry2):
            for kk in range(8):
                sl = pl.ds(kk * 16, 16)
                acc_v[r, sl] = acc_v[r, sl] + rows_v[r, sl]
            return carry2
        lax.fori_loop(0, 128, add_row, 0)
        pltpu.sync_copy(acc_v, out_hbm.at[pl.ds(base, 128)])
        return carry
    lax.fori_loop(0, 5, chunk, 0)


def _sc_embed(ann3, base_flat, projT):
    kern = pl.kernel(
        _sc_embed_body,
        out_type=jax.ShapeDtypeStruct((NP * 2, H), F32),
        mesh=_mesh(),
        scratch_types=[
            pltpu.VMEM((5, 128), I32),
            pltpu.VMEM((128, H), F32),
            pltpu.VMEM((128, H), F32),
            pltpu.SemaphoreType.DMA,
        ],
    )
    return kern(ann3, base_flat, projT)


def _sc_edge_gather_body(nf_hbm, s0_hbm, s1_hbm, d0_hbm, d1_hbm,
                         gs0_hbm, gs1_hbm, gd0_hbm, gd1_hbm,
                         idx_v, buf, buf2, sem, sem2):
    c = lax.axis_index("c")
    s = lax.axis_index("s")
    w = c * 16 + s
    e0 = w * 5120

    for (src_idx_hbm, out_hbm) in ((s0_hbm, gs0_hbm), (s1_hbm, gs1_hbm),
                                   (d0_hbm, gd0_hbm), (d1_hbm, gd1_hbm)):
        pltpu.sync_copy(src_idx_hbm.at[w], idx_v)
        pltpu.async_copy(nf_hbm.at[idx_v.at[0]], buf, sem)
        pltpu.async_copy(nf_hbm.at[idx_v.at[1]], buf2, sem2)

        def body(jj, carry):
            j = 2 * jj
            pltpu.make_async_copy(nf_hbm.at[idx_v.at[j]], buf, sem).wait()
            pltpu.sync_copy(buf, out_hbm.at[pl.ds(e0 + j * 128, 128)])

            @pl.when(jj < 19)
            def _():
                pltpu.async_copy(nf_hbm.at[idx_v.at[j + 2]], buf, sem)
            pltpu.make_async_copy(nf_hbm.at[idx_v.at[j + 1]], buf2,
                                  sem2).wait()
            pltpu.sync_copy(buf2, out_hbm.at[pl.ds(e0 + (j + 1) * 128, 128)])

            @pl.when(jj < 19)
            def _():
                pltpu.async_copy(nf_hbm.at[idx_v.at[j + 3]], buf2, sem2)
            return carry
        lax.fori_loop(0, 20, body, 0)


def _sc_edge_gather(nf_flat, s0, s1, d0, d1):
    kern = pl.kernel(
        _sc_edge_gather_body,
        out_type=(jax.ShapeDtypeStruct((EP, H), F32),
                  jax.ShapeDtypeStruct((EP, H), F32),
                  jax.ShapeDtypeStruct((EP, H), F32),
                  jax.ShapeDtypeStruct((EP, H), F32)),
        mesh=_mesh(),
        scratch_types=[
            pltpu.VMEM((40, 128), I32),
            pltpu.VMEM((128, H), F32),
            pltpu.VMEM((128, H), F32),
            pltpu.SemaphoreType.DMA,
            pltpu.SemaphoreType.DMA,
        ],
    )
    return kern(nf_flat, s0, s1, d0, d1)


B_E = 1024


def _edge_mlp_body(gs0_ref, gs1_ref, gd0_ref, gd1_ref, W1LT_ref, W1RT_ref,
                   b1_ref, W2T_ref, b2_ref, Wn1T_ref, b_ref):
    for t, (gs_ref, gd_ref) in enumerate(((gs0_ref, gd0_ref),
                                          (gs1_ref, gd1_ref))):
        h = (jnp.dot(gs_ref[...], W1LT_ref[...], preferred_element_type=F32)
             + jnp.dot(gd_ref[...], W1RT_ref[...], preferred_element_type=F32)
             + b1_ref[...])
        h = _elu(h)
        a = _elu(jnp.dot(h, W2T_ref[...], preferred_element_type=F32)
                 + b2_ref[...])
        b_ref[t, :, :] = jnp.dot(a, Wn1T_ref[...], preferred_element_type=F32)


def _tc_edge_mlp(gs0, gs1, gd0, gd1, W1LT, W1RT, b1, W2T, b2, Wn1T):
    nb = EP // B_E
    edge_spec = pl.BlockSpec((B_E, H), lambda i: (i, 0))
    return pl.pallas_call(
        _edge_mlp_body,
        grid=(nb,),
        in_specs=[
            edge_spec, edge_spec, edge_spec, edge_spec,
            pl.BlockSpec((H, 2 * H), lambda i: (0, 0)),
            pl.BlockSpec((H, 2 * H), lambda i: (0, 0)),
            pl.BlockSpec((1, 2 * H), lambda i: (0, 0)),
            pl.BlockSpec((2 * H, 2 * H), lambda i: (0, 0)),
            pl.BlockSpec((1, 2 * H), lambda i: (0, 0)),
            pl.BlockSpec((2 * H, H), lambda i: (0, 0)),
        ],
        out_specs=pl.BlockSpec((2, B_E, H), lambda i: (0, i, 0)),
        out_shape=jax.ShapeDtypeStruct((2, EP, H), F32),
    )(gs0, gs1, gd0, gd1, W1LT, W1RT, b1, W2T, b2, Wn1T)


def _sc_scatter_body(b_hbm, dst_hbm, ones_hbm, zs_hbm,
                     sums_hbm, cnt_hbm, idx, rows, ones_v, acc_sh, sem):
    c = lax.axis_index("c")
    s = lax.axis_index("s")
    pltpu.sync_copy(zs_hbm.at[pl.ds(s * 640, 640)],
                    acc_sh.at[pl.ds(s * 640, 640)])
    pltpu.sync_copy(dst_hbm.at[s], idx)
    pltpu.sync_copy(ones_hbm, ones_v)
    plsc.subcore_barrier()
    e0 = s * 10240

    def body(j, carry):
        pltpu.sync_copy(b_hbm.at[c, pl.ds(e0 + j * 128, 128)], rows)
        pltpu.sync_copy(rows, acc_sh.at[idx.at[j]], add=True)
        return carry
    lax.fori_loop(0, 80, body, 0)
    plsc.subcore_barrier()
    pltpu.sync_copy(acc_sh.at[pl.ds(s * 640, 640)],
                    sums_hbm.at[c, pl.ds(s * 640, 640)])
    pltpu.sync_copy(zs_hbm.at[pl.ds(s * 640, 640)],
                    acc_sh.at[pl.ds(s * 640, 640)])
    plsc.subcore_barrier()

    def cbody(j, carry):
        pltpu.sync_copy(ones_v, acc_sh.at[idx.at[j]], add=True)
        return carry
    lax.fori_loop(c * 40, (c + 1) * 40, cbody, 0)
    plsc.subcore_barrier()
    pltpu.sync_copy(acc_sh.at[pl.ds(s * 640, 640)],
                    cnt_hbm.at[c, pl.ds(s * 640, 640)])


def _sc_scatter(b2, dst16, ones128, zs):
    kern = pl.kernel(
        _sc_scatter_body,
        out_type=(jax.ShapeDtypeStruct((2, NP, H), F32),
                  jax.ShapeDtypeStruct((2, NP, H), F32)),
        mesh=_mesh(),
        scratch_types=[
            pltpu.VMEM((80, 128), I32),
            pltpu.VMEM((128, H), F32),
            pltpu.VMEM((128, H), F32),
            pltpu.VMEM_SHARED((NP, H), F32),
            pltpu.SemaphoreType.DMA,
        ],
    )
    return kern(b2, dst16, ones128, zs)


B_N = 1024


def _epilogue_body(sums_ref, cnt_ref, nb1_ref, Wn2T_ref, nb2_ref, WihT_ref,
                   WhhT_ref, bihh_ref, hWT_ref, hb_ref, out_ref):
    cvec = cnt_ref[0] + cnt_ref[1]
    den = jnp.maximum(cvec[:, 0:1], 1.0)
    xs = []
    for t in range(2):
        x1 = sums_ref[t] / den + nb1_ref[...]
        nf2 = _elu(jnp.dot(_elu(x1), Wn2T_ref[...],
                           preferred_element_type=F32) + nb2_ref[...])
        xs.append(nf2)
    bihh = bihh_ref[...]
    g1 = jnp.dot(xs[0], WihT_ref[...], preferred_element_type=F32) + bihh
    c1 = _sigmoid(g1[:, 0:H]) * jnp.tanh(g1[:, 2 * H:3 * H])
    h1 = _sigmoid(g1[:, 3 * H:4 * H]) * jnp.tanh(c1)
    g2 = (jnp.dot(xs[1], WihT_ref[...], preferred_element_type=F32)
          + jnp.dot(h1, WhhT_ref[...], preferred_element_type=F32) + bihh)
    c2 = (_sigmoid(g2[:, H:2 * H]) * c1
          + _sigmoid(g2[:, 0:H]) * jnp.tanh(g2[:, 2 * H:3 * H]))
    h2 = _sigmoid(g2[:, 3 * H:4 * H]) * jnp.tanh(c2)
    out_ref[...] = (jnp.dot(h2, hWT_ref[...], preferred_element_type=F32)
                    + hb_ref[...])


def _tc_epilogue(sums, cnt, nb1, Wn2T, nb2, WihT, WhhT, bihh, hWT, hb):
    nb = NP // B_N
    return pl.pallas_call(
        _epilogue_body,
        grid=(nb,),
        in_specs=[
            pl.BlockSpec((2, B_N, H), lambda i: (0, i, 0)),
            pl.BlockSpec((2, B_N, H), lambda i: (0, i, 0)),
            pl.BlockSpec((1, H), lambda i: (0, 0)),
            pl.BlockSpec((H, H), lambda i: (0, 0)),
            pl.BlockSpec((1, H), lambda i: (0, 0)),
            pl.BlockSpec((H, 4 * H), lambda i: (0, 0)),
            pl.BlockSpec((H, 4 * H), lambda i: (0, 0)),
            pl.BlockSpec((1, 4 * H), lambda i: (0, 0)),
            pl.BlockSpec((H, H), lambda i: (0, 0)),
            pl.BlockSpec((1, H), lambda i: (0, 0)),
        ],
        out_specs=pl.BlockSpec((B_N, H), lambda i: (i, 0)),
        out_shape=jax.ShapeDtypeStruct((NP, H), F32),
    )(sums, cnt, nb1, Wn2T, nb2, WihT, WhhT, bihh, hWT, hb)


def kernel(edge_index, input_time, input_events1, event_annotation, pos_vec,
           emb_table, W_in, b_in, attn_W1, attn_b1, attn_W2, attn_b2,
           node_W1, node_b1, node_W2, node_b2, lstm_Wih, lstm_bih, lstm_Whh,
           lstm_bhh, model_W, model_b, time_W, time_b, event_W, event_b):
    N, T = input_time.shape
    E = edge_index.shape[1]

    W_emb = jnp.pad(W_in[:, :H - 1], ((0, 0), (0, 1)))
    wev2 = W_in[:, H - 1].reshape(1, H)
    WteT = W_in[:, H:].T
    W1LT = attn_W1[:, :H].T
    W1RT = attn_W1[:, H:].T
    emb_pad = jnp.pad(emb_table, ((0, 1024 - emb_table.shape[0]), (0, 1)))
    headW = jnp.concatenate(
        [model_W, time_W, event_W, jnp.zeros((H - 3, H), F32)], axis=0)
    headb = jnp.concatenate(
        [model_b, time_b, event_b, jnp.zeros((H - 3,), F32)]).reshape(1, H)

    src = jnp.concatenate(
        [edge_index[0].astype(I32), jnp.zeros((EP - E,), I32)])
    dst = jnp.concatenate(
        [edge_index[1].astype(I32), jnp.full((EP - E,), N, I32)])
    time_p = jnp.pad(input_time, ((0, NP - N), (0, 0)))
    ev_p = jnp.pad(input_events1, ((0, NP - N), (0, 0)))
    ann3 = jnp.pad(event_annotation.astype(I32),
                   ((0, NP - N), (0, 0))).reshape(32, 5, 128)

    projT, base = _tc_prologue(time_p, ev_p, pos_vec.reshape(1, H), emb_pad,
                               W_emb.T, wev2, WteT, b_in.reshape(1, H))
    nf_flat = _sc_embed(ann3, base.reshape(NP * 2, H), projT)

    gs0, gs1, gd0, gd1 = _sc_edge_gather(
        nf_flat,
        (src * 2).reshape(32, 40, 128), (src * 2 + 1).reshape(32, 40, 128),
        (dst * 2).reshape(32, 40, 128), (dst * 2 + 1).reshape(32, 40, 128))

    b2 = _tc_edge_mlp(gs0, gs1, gd0, gd1, W1LT, W1RT,
                      attn_b1.reshape(1, 2 * H),
                      attn_W2.T, attn_b2.reshape(1, 2 * H), node_W1.T)

    sums, cnt = _sc_scatter(b2, dst.reshape(16, 80, 128),
                            jnp.ones((128, H), F32), jnp.zeros((NP, H), F32))

    heads = _tc_epilogue(sums, cnt, node_b1.reshape(1, H), node_W2.T,
                         node_b2.reshape(1, H), lstm_Wih.T, lstm_Whh.T,
                         (lstm_bih + lstm_bhh).reshape(1, 4 * H),
                         headW.T, headb)
    return heads[:N, 0:1], heads[:N, 1:2], heads[:N, 2:3]
